# baseline (speedup 1.0000x reference)
import functools

import jax
import jax.numpy as jnp
from jax import lax
from jax.experimental import pallas as pl
from jax.experimental.pallas import tpu as pltpu
from jax.experimental.pallas import tpu_sc as plsc

N = 10000
E = 320000
IN, H1, H2, OUT = 128, 64, 64, 32

NC, NS = 2, 16
NW = NC * NS
PER_TILE = E // NW
CHUNK = 80
ITERS = PER_TILE // CHUNK
ROWS_PER_TILE = N // NS

_mesh = lambda: plsc.VectorSubcoreMesh(
    core_axis_name="c", subcore_axis_name="s", num_cores=NC, num_subcores=NS)

_sc_params = lambda: pltpu.CompilerParams(use_tc_tiling_on_sc=False)


def _sc_degree(dst3, zeros16):
    @functools.partial(
        pl.kernel,
        out_type=jax.ShapeDtypeStruct((NC, N, 16), jnp.float32),
        mesh=_mesh(),
        compiler_params=_sc_params(),
        scratch_types=[
            pltpu.VMEM((ITERS, CHUNK), jnp.int32),
            pltpu.VMEM((CHUNK, 16), jnp.float32),
            pltpu.VMEM_SHARED((N, 16), jnp.float32),
            pltpu.SemaphoreType.DMA,
        ],
    )
    def k(dst3_hbm, z_hbm, out_hbm, didx_v, ones_v, accum, ssem):
        cid = lax.axis_index("c")
        sid = lax.axis_index("s")
        wid = cid * NS + sid
        for i in range(CHUNK):
            ones_v[i, :] = jnp.full((16,), 1.0, jnp.float32)
        pltpu.sync_copy(dst3_hbm.at[wid], didx_v)

        @pl.when(sid == 0)
        def _init():
            pltpu.sync_copy(z_hbm, accum)

        plsc.subcore_barrier()

        def body(i, carry):
            pltpu.async_copy(ones_v, accum.at[didx_v.at[i]], ssem, add=True)

            @pl.when(i >= 8)
            def _throttle():
                pltpu.make_async_copy(
                    ones_v, accum.at[didx_v.at[i]], ssem).wait()

            return carry

        lax.fori_loop(0, ITERS, body, 0)
        for _ in range(8):
            pltpu.make_async_copy(ones_v, accum.at[didx_v.at[0]], ssem).wait()
        plsc.subcore_barrier()

        @pl.when(sid == 0)
        def _writeback():
            pltpu.sync_copy(accum, out_hbm.at[cid])

    return k(dst3, zeros16)


NB = 4
PFD = 3
HH = H1 // 2
E_TILE = E // NS
ITERS2 = E_TILE // CHUNK


def _sc_propagate(src2, dst2, y_lo, y_hi):
    @functools.partial(
        pl.kernel,
        out_type=jax.ShapeDtypeStruct((NC, N, HH), jnp.float32),
        mesh=_mesh(),
        compiler_params=_sc_params(),
        scratch_types=[
            pltpu.VMEM((ITERS2, CHUNK), jnp.int32),
            pltpu.VMEM((ITERS2, CHUNK), jnp.int32),
            pltpu.VMEM((NB, CHUNK, HH), jnp.float32),
            pltpu.VMEM_SHARED((N, HH), jnp.float32),
            pltpu.SemaphoreType.DMA((NB,)),
            pltpu.SemaphoreType.DMA((NB,)),
        ],
    )
    def k(src2_hbm, dst2_hbm, ylo_hbm, yhi_hbm, out_hbm, sidx_v, didx_v,
          rows_v, accum, gsem, ssem):
        cid = lax.axis_index("c")
        sid = lax.axis_index("s")

        pltpu.sync_copy(src2_hbm.at[sid], sidx_v)
        pltpu.sync_copy(dst2_hbm.at[sid], didx_v)

        def pipeline(y_hbm):
            @pl.when(sid == 0)
            def _init():
                pltpu.sync_copy(y_hbm, accum)

            for j in range(PFD):
                pltpu.async_copy(y_hbm.at[sidx_v.at[j]], rows_v.at[j],
                                 gsem.at[j])
            plsc.subcore_barrier()

            def body(i, carry):
                @pl.when(i + PFD < ITERS2)
                def _prefetch():
                    bn = lax.rem(i + PFD, NB)

                    @pl.when(i + PFD >= NB)
                    def _slot_free():
                        pltpu.make_async_copy(
                            rows_v.at[bn], accum.at[didx_v.at[i]], ssem.at[bn]
                        ).wait()

                    pltpu.async_copy(
                        y_hbm.at[sidx_v.at[i + PFD]],
                        rows_v.at[bn], gsem.at[bn])

                b = lax.rem(i, NB)
                pltpu.make_async_copy(
                    y_hbm.at[sidx_v.at[i]], rows_v.at[b], gsem.at[b]).wait()
                pltpu.async_copy(rows_v.at[b], accum.at[didx_v.at[i]],
                                 ssem.at[b], add=True)
                return carry

            lax.fori_loop(0, ITERS2, body, 0)
            for j in range(NB):
                pltpu.make_async_copy(
                    rows_v.at[j], accum.at[didx_v.at[0]], ssem.at[j]).wait()
            plsc.subcore_barrier()

        @pl.when(cid == 0)
        def _lo():
            pipeline(ylo_hbm)

        @pl.when(cid == 1)
        def _hi():
            pipeline(yhi_hbm)

        @pl.when(sid == 0)
        def _writeback():
            pltpu.sync_copy(accum, out_hbm.at[cid])

    return k(src2, dst2, y_lo, y_hi)


_BLK = 1000


def _tc_matmul(x, W1):
    def body(x_ref, w_ref, o_ref):
        o_ref[...] = jnp.dot(x_ref[...], w_ref[...],
                             preferred_element_type=jnp.float32)

    return pl.pallas_call(
        body,
        grid=(N // _BLK,),
        in_specs=[
            pl.BlockSpec((_BLK, IN), lambda i: (i, 0)),
            pl.BlockSpec((IN, H1), lambda i: (0, 0)),
        ],
        out_specs=pl.BlockSpec((_BLK, H1), lambda i: (i, 0)),
        out_shape=jax.ShapeDtypeStruct((N, H1), jnp.float32),
    )(x, W1)


def _tc_scale(dp, xw1):
    def body(dp_ref, xw_ref, ylo_ref, yhi_ref, dinv_ref):
        deg = dp_ref[0, :, 0:1] + dp_ref[1, :, 0:1] + 1.0
        dinv = lax.rsqrt(deg)
        y = dinv * xw_ref[...]
        ylo_ref[...] = y[:, :HH]
        yhi_ref[...] = y[:, HH:]
        dinv_ref[...] = jnp.broadcast_to(dinv, (_BLK, 16))

    return pl.pallas_call(
        body,
        grid=(N // _BLK,),
        in_specs=[
            pl.BlockSpec((NC, _BLK, 16), lambda i: (0, i, 0)),
            pl.BlockSpec((_BLK, H1), lambda i: (i, 0)),
        ],
        out_specs=[
            pl.BlockSpec((_BLK, HH), lambda i: (i, 0)),
            pl.BlockSpec((_BLK, HH), lambda i: (i, 0)),
            pl.BlockSpec((_BLK, 16), lambda i: (i, 0)),
        ],
        out_shape=[
            jax.ShapeDtypeStruct((N, HH), jnp.float32),
            jax.ShapeDtypeStruct((N, HH), jnp.float32),
            jax.ShapeDtypeStruct((N, 16), jnp.float32),
        ],
    )(dp, xw1)


def _tc_mid(p, dinv16, W2, b1):
    def body(p_ref, dinv_ref, w_ref, b_ref, ylo_ref, yhi_ref):
        dinv = dinv_ref[:, 0:1]
        h1 = dinv * jnp.concatenate([p_ref[0], p_ref[1]], axis=1) + b_ref[0, :]
        y2 = dinv * jnp.dot(h1, w_ref[...], preferred_element_type=jnp.float32)
        ylo_ref[...] = y2[:, :HH]
        yhi_ref[...] = y2[:, HH:]

    return pl.pallas_call(
        body,
        grid=(N // _BLK,),
        in_specs=[
            pl.BlockSpec((NC, _BLK, HH), lambda i: (0, i, 0)),
            pl.BlockSpec((_BLK, 16), lambda i: (i, 0)),
            pl.BlockSpec((H1, H2), lambda i: (0, 0)),
            pl.BlockSpec((1, H1), lambda i: (0, 0)),
        ],
        out_specs=[
            pl.BlockSpec((_BLK, HH), lambda i: (i, 0)),
            pl.BlockSpec((_BLK, HH), lambda i: (i, 0)),
        ],
        out_shape=[
            jax.ShapeDtypeStruct((N, HH), jnp.float32),
            jax.ShapeDtypeStruct((N, HH), jnp.float32),
        ],
    )(p, dinv16, W2, b1.reshape(1, H1))


def _tc_heads(r, dinv16, b2, Wmu, bmu, Wlv, blv):
    def body(r_ref, dinv_ref, b2_ref, wmu_ref, bmu_ref, wlv_ref, blv_ref,
             mu_ref, lv_ref):
        dinv = dinv_ref[:, 0:1]
        h2 = dinv * jnp.concatenate([r_ref[0], r_ref[1]], axis=1) + b2_ref[0, :]
        mu_ref[...] = jnp.dot(h2, wmu_ref[...], preferred_element_type=jnp.float32) + bmu_ref[0, :]
        lv_ref[...] = jnp.dot(h2, wlv_ref[...], preferred_element_type=jnp.float32) + blv_ref[0, :]

    return pl.pallas_call(
        body,
        grid=(N // _BLK,),
        in_specs=[
            pl.BlockSpec((NC, _BLK, HH), lambda i: (0, i, 0)),
            pl.BlockSpec((_BLK, 16), lambda i: (i, 0)),
            pl.BlockSpec((1, H2), lambda i: (0, 0)),
            pl.BlockSpec((H2, OUT), lambda i: (0, 0)),
            pl.BlockSpec((1, OUT), lambda i: (0, 0)),
            pl.BlockSpec((H2, OUT), lambda i: (0, 0)),
            pl.BlockSpec((1, OUT), lambda i: (0, 0)),
        ],
        out_specs=[
            pl.BlockSpec((_BLK, OUT), lambda i: (i, 0)),
            pl.BlockSpec((_BLK, OUT), lambda i: (i, 0)),
        ],
        out_shape=[
            jax.ShapeDtypeStruct((N, OUT), jnp.float32),
            jax.ShapeDtypeStruct((N, OUT), jnp.float32),
        ],
    )(r, dinv16, b2.reshape(1, H2), Wmu, bmu.reshape(1, OUT),
      Wlv, blv.reshape(1, OUT))


def kernel(x, edge_index, W1, b1, W2, b2, Wmu, bmu, Wlv, blv):
    src2 = lax.optimization_barrier(edge_index[0]).reshape(NS, ITERS2, CHUNK)
    dst = edge_index[1]
    dst3 = dst.reshape(NW, ITERS, CHUNK)
    dst2 = dst.reshape(NS, ITERS2, CHUNK)
    xw1 = _tc_matmul(x, W1)
    dp = _sc_degree(dst3, jnp.zeros((N, 16), jnp.float32))
    y1lo, y1hi, dinv16 = _tc_scale(dp, xw1)
    p = _sc_propagate(src2, dst2, y1lo, y1hi)
    y2lo, y2hi = _tc_mid(p, dinv16, W2, b1)
    r = _sc_propagate(src2, dst2, y2lo, y2hi)
    mu, log_var = _tc_heads(r, dinv16, b2, Wmu, bmu, Wlv, blv)
    return (mu, log_var)


# speedup vs baseline: 1.1569x; 1.1569x over previous
import functools

import jax
import jax.numpy as jnp
from jax import lax
from jax.experimental import pallas as pl
from jax.experimental.pallas import tpu as pltpu
from jax.experimental.pallas import tpu_sc as plsc

N = 10000
E = 320000
IN, H1, H2, OUT = 128, 64, 64, 32

NC, NS = 2, 16
NW = NC * NS
PER_TILE = E // NW
CHUNK = 80
ITERS = PER_TILE // CHUNK
ROWS_PER_TILE = N // NS

_mesh = lambda: plsc.VectorSubcoreMesh(
    core_axis_name="c", subcore_axis_name="s", num_cores=NC, num_subcores=NS)

_sc_params = lambda: pltpu.CompilerParams(use_tc_tiling_on_sc=False)


def _sc_degree(dst3, zeros16):
    @functools.partial(
        pl.kernel,
        out_type=jax.ShapeDtypeStruct((NC, N, 16), jnp.float32),
        mesh=_mesh(),
        compiler_params=_sc_params(),
        scratch_types=[
            pltpu.VMEM((ITERS, CHUNK), jnp.int32),
            pltpu.VMEM((CHUNK, 16), jnp.float32),
            pltpu.VMEM_SHARED((N, 16), jnp.float32),
            pltpu.SemaphoreType.DMA,
        ],
    )
    def k(dst3_hbm, z_hbm, out_hbm, didx_v, ones_v, accum, ssem):
        cid = lax.axis_index("c")
        sid = lax.axis_index("s")
        wid = cid * NS + sid
        for i in range(CHUNK):
            ones_v[i, :] = jnp.full((16,), 1.0, jnp.float32)
        pltpu.sync_copy(dst3_hbm.at[wid], didx_v)

        @pl.when(sid == 0)
        def _init():
            pltpu.sync_copy(z_hbm, accum)

        plsc.subcore_barrier()

        def body(i, carry):
            pltpu.async_copy(ones_v, accum.at[didx_v.at[i]], ssem, add=True)

            @pl.when(i >= 8)
            def _throttle():
                pltpu.make_async_copy(
                    ones_v, accum.at[didx_v.at[i]], ssem).wait()

            return carry

        lax.fori_loop(0, ITERS, body, 0)
        for _ in range(8):
            pltpu.make_async_copy(ones_v, accum.at[didx_v.at[0]], ssem).wait()
        plsc.subcore_barrier()

        @pl.when(sid == 0)
        def _writeback():
            pltpu.sync_copy(accum, out_hbm.at[cid])

    return k(dst3, zeros16)


NB = 4
PFD = 3


def _sc_propagate(src3, dst3, y):
    @functools.partial(
        pl.kernel,
        out_type=jax.ShapeDtypeStruct((NC, N, H1), jnp.float32),
        mesh=_mesh(),
        compiler_params=_sc_params(),
        scratch_types=[
            pltpu.VMEM((ITERS, CHUNK), jnp.int32),
            pltpu.VMEM((ITERS, CHUNK), jnp.int32),
            pltpu.VMEM((NB, CHUNK, H1), jnp.float32),
            pltpu.VMEM_SHARED((N, H1), jnp.float32),
            pltpu.SemaphoreType.DMA((NB,)),
            pltpu.SemaphoreType.DMA((NB,)),
        ],
    )
    def k(src3_hbm, dst3_hbm, y_hbm, out_hbm, sidx_v, didx_v, rows_v, accum,
          gsem, ssem):
        cid = lax.axis_index("c")
        sid = lax.axis_index("s")
        wid = cid * NS + sid

        @pl.when(sid == 0)
        def _init():
            pltpu.sync_copy(y_hbm, accum)

        pltpu.sync_copy(src3_hbm.at[wid], sidx_v)
        pltpu.sync_copy(dst3_hbm.at[wid], didx_v)
        for j in range(PFD):
            pltpu.async_copy(y_hbm.at[sidx_v.at[j]], rows_v.at[j], gsem.at[j])
        plsc.subcore_barrier()

        def body(i, carry):
            @pl.when(i + PFD < ITERS)
            def _prefetch():
                bn = lax.rem(i + PFD, NB)

                @pl.when(i + PFD >= NB)
                def _slot_free():
                    pltpu.make_async_copy(
                        rows_v.at[bn], accum.at[didx_v.at[i]], ssem.at[bn]
                    ).wait()

                pltpu.async_copy(
                    y_hbm.at[sidx_v.at[i + PFD]],
                    rows_v.at[bn], gsem.at[bn])

            b = lax.rem(i, NB)
            pltpu.make_async_copy(
                y_hbm.at[sidx_v.at[i]], rows_v.at[b], gsem.at[b]).wait()
            pltpu.async_copy(rows_v.at[b], accum.at[didx_v.at[i]],
                             ssem.at[b], add=True)
            return carry

        lax.fori_loop(0, ITERS, body, 0)
        for j in range(NB):
            pltpu.make_async_copy(
                rows_v.at[j], accum.at[didx_v.at[0]], ssem.at[j]).wait()
        plsc.subcore_barrier()

        @pl.when(sid == 0)
        def _writeback():
            pltpu.sync_copy(accum, out_hbm.at[cid])

    return k(src3, dst3, y)


_BLK = 1000


def _tc_matmul(x, W1):
    def body(x_ref, w_ref, o_ref):
        o_ref[...] = jnp.dot(x_ref[...], w_ref[...],
                             preferred_element_type=jnp.float32)

    return pl.pallas_call(
        body,
        grid=(N // _BLK,),
        in_specs=[
            pl.BlockSpec((_BLK, IN), lambda i: (i, 0)),
            pl.BlockSpec((IN, H1), lambda i: (0, 0)),
        ],
        out_specs=pl.BlockSpec((_BLK, H1), lambda i: (i, 0)),
        out_shape=jax.ShapeDtypeStruct((N, H1), jnp.float32),
    )(x, W1)


def _tc_scale(dp, xw1):
    def body(dp_ref, xw_ref, y_ref, dinv_ref):
        deg = dp_ref[0, :, 0:1] + dp_ref[1, :, 0:1] + 1.0
        dinv = lax.rsqrt(deg)
        y_ref[...] = dinv * xw_ref[...]
        dinv_ref[...] = jnp.broadcast_to(dinv, (_BLK, 16))

    return pl.pallas_call(
        body,
        grid=(N // _BLK,),
        in_specs=[
            pl.BlockSpec((NC, _BLK, 16), lambda i: (0, i, 0)),
            pl.BlockSpec((_BLK, H1), lambda i: (i, 0)),
        ],
        out_specs=[
            pl.BlockSpec((_BLK, H1), lambda i: (i, 0)),
            pl.BlockSpec((_BLK, 16), lambda i: (i, 0)),
        ],
        out_shape=[
            jax.ShapeDtypeStruct((N, H1), jnp.float32),
            jax.ShapeDtypeStruct((N, 16), jnp.float32),
        ],
    )(dp, xw1)


def _tc_mid(p, y1, dinv16, W2, b1):
    def body(p_ref, y_ref, dinv_ref, w_ref, b_ref, out_ref):
        dinv = dinv_ref[:, 0:1]
        h1 = dinv * (p_ref[0] + p_ref[1] - y_ref[...]) + b_ref[0, :]
        out_ref[...] = dinv * jnp.dot(h1, w_ref[...], preferred_element_type=jnp.float32)

    return pl.pallas_call(
        body,
        grid=(N // _BLK,),
        in_specs=[
            pl.BlockSpec((NC, _BLK, H1), lambda i: (0, i, 0)),
            pl.BlockSpec((_BLK, H1), lambda i: (i, 0)),
            pl.BlockSpec((_BLK, 16), lambda i: (i, 0)),
            pl.BlockSpec((H1, H2), lambda i: (0, 0)),
            pl.BlockSpec((1, H1), lambda i: (0, 0)),
        ],
        out_specs=pl.BlockSpec((_BLK, H2), lambda i: (i, 0)),
        out_shape=jax.ShapeDtypeStruct((N, H2), jnp.float32),
    )(p, y1, dinv16, W2, b1.reshape(1, H1))


def _tc_heads(r, y2, dinv16, b2, Wmu, bmu, Wlv, blv):
    def body(r_ref, y_ref, dinv_ref, b2_ref, wmu_ref, bmu_ref, wlv_ref, blv_ref,
             mu_ref, lv_ref):
        dinv = dinv_ref[:, 0:1]
        h2 = dinv * (r_ref[0] + r_ref[1] - y_ref[...]) + b2_ref[0, :]
        mu_ref[...] = jnp.dot(h2, wmu_ref[...], preferred_element_type=jnp.float32) + bmu_ref[0, :]
        lv_ref[...] = jnp.dot(h2, wlv_ref[...], preferred_element_type=jnp.float32) + blv_ref[0, :]

    return pl.pallas_call(
        body,
        grid=(N // _BLK,),
        in_specs=[
            pl.BlockSpec((NC, _BLK, H2), lambda i: (0, i, 0)),
            pl.BlockSpec((_BLK, H2), lambda i: (i, 0)),
            pl.BlockSpec((_BLK, 16), lambda i: (i, 0)),
            pl.BlockSpec((1, H2), lambda i: (0, 0)),
            pl.BlockSpec((H2, OUT), lambda i: (0, 0)),
            pl.BlockSpec((1, OUT), lambda i: (0, 0)),
            pl.BlockSpec((H2, OUT), lambda i: (0, 0)),
            pl.BlockSpec((1, OUT), lambda i: (0, 0)),
        ],
        out_specs=[
            pl.BlockSpec((_BLK, OUT), lambda i: (i, 0)),
            pl.BlockSpec((_BLK, OUT), lambda i: (i, 0)),
        ],
        out_shape=[
            jax.ShapeDtypeStruct((N, OUT), jnp.float32),
            jax.ShapeDtypeStruct((N, OUT), jnp.float32),
        ],
    )(r, y2, dinv16, b2.reshape(1, H2), Wmu, bmu.reshape(1, OUT),
      Wlv, blv.reshape(1, OUT))


def kernel(x, edge_index, W1, b1, W2, b2, Wmu, bmu, Wlv, blv):
    src3 = lax.optimization_barrier(edge_index[0]).reshape(NW, ITERS, CHUNK)
    dst3 = edge_index[1].reshape(NW, ITERS, CHUNK)
    xw1 = _tc_matmul(x, W1)
    dp = _sc_degree(dst3, jnp.zeros((N, 16), jnp.float32))
    y1, dinv16 = _tc_scale(dp, xw1)
    p = _sc_propagate(src3, dst3, y1)
    y2 = _tc_mid(p, y1, dinv16, W2, b1)
    r = _sc_propagate(src3, dst3, y2)
    mu, log_var = _tc_heads(r, y2, dinv16, b2, Wmu, bmu, Wlv, blv)
    return (mu, log_var)


# speedup vs baseline: 1.1583x; 1.0011x over previous
import functools

import jax
import jax.numpy as jnp
from jax import lax
from jax.experimental import pallas as pl
from jax.experimental.pallas import tpu as pltpu
from jax.experimental.pallas import tpu_sc as plsc

N = 10000
E = 320000
IN, H1, H2, OUT = 128, 64, 64, 32

NC, NS = 2, 16
NW = NC * NS
PER_TILE = E // NW
CHUNK = 80
ITERS = PER_TILE // CHUNK

_mesh = lambda: plsc.VectorSubcoreMesh(
    core_axis_name="c", subcore_axis_name="s", num_cores=NC, num_subcores=NS)

_sc_params = lambda: pltpu.CompilerParams(use_tc_tiling_on_sc=False)


def _sc_degree(dst3, zeros16):
    @functools.partial(
        pl.kernel,
        out_type=jax.ShapeDtypeStruct((NC, N, 16), jnp.float32),
        mesh=_mesh(),
        compiler_params=_sc_params(),
        scratch_types=[
            pltpu.VMEM((ITERS, CHUNK), jnp.int32),
            pltpu.VMEM((CHUNK, 16), jnp.float32),
            pltpu.VMEM_SHARED((N, 16), jnp.float32),
            pltpu.SemaphoreType.DMA,
        ],
    )
    def k(dst3_hbm, z_hbm, out_hbm, didx_v, ones_v, accum, ssem):
        cid = lax.axis_index("c")
        sid = lax.axis_index("s")
        wid = cid * NS + sid
        for i in range(CHUNK):
            ones_v[i, :] = jnp.full((16,), 1.0, jnp.float32)
        pltpu.sync_copy(dst3_hbm.at[wid], didx_v)

        @pl.when(sid == 0)
        def _init():
            pltpu.sync_copy(z_hbm, accum)

        plsc.subcore_barrier()

        def body(i, carry):
            pltpu.async_copy(ones_v, accum.at[didx_v.at[i]], ssem, add=True)

            @pl.when(i >= 8)
            def _throttle():
                pltpu.make_async_copy(
                    ones_v, accum.at[didx_v.at[i]], ssem).wait()

            return carry

        lax.fori_loop(0, ITERS, body, 0)
        for _ in range(8):
            pltpu.make_async_copy(ones_v, accum.at[didx_v.at[0]], ssem).wait()
        plsc.subcore_barrier()

        @pl.when(sid == 0)
        def _writeback():
            pltpu.sync_copy(accum, out_hbm.at[cid])

    return k(dst3, zeros16)


NB = 4
PFD = 3


def _sc_propagate(src3, dst3, y):
    @functools.partial(
        pl.kernel,
        out_type=jax.ShapeDtypeStruct((NC, N, H1), jnp.float32),
        mesh=_mesh(),
        compiler_params=_sc_params(),
        scratch_types=[
            pltpu.VMEM((ITERS, CHUNK), jnp.int32),
            pltpu.VMEM((ITERS, CHUNK), jnp.int32),
            pltpu.VMEM((NB, CHUNK, H1), jnp.float32),
            pltpu.VMEM_SHARED((N, H1), jnp.float32),
            pltpu.SemaphoreType.DMA((NB,)),
            pltpu.SemaphoreType.DMA((NB,)),
        ],
    )
    def k(src3_hbm, dst3_hbm, y_hbm, out_hbm, sidx_v, didx_v, rows_v, accum,
          gsem, ssem):
        cid = lax.axis_index("c")
        sid = lax.axis_index("s")
        wid = cid * NS + sid

        @pl.when(sid == 0)
        def _init():
            pltpu.sync_copy(y_hbm, accum)

        pltpu.sync_copy(src3_hbm.at[wid], sidx_v)
        pltpu.sync_copy(dst3_hbm.at[wid], didx_v)
        for j in range(PFD):
            pltpu.async_copy(y_hbm.at[sidx_v.at[j]], rows_v.at[j], gsem.at[j])
        plsc.subcore_barrier()

        def body(i, carry):
            @pl.when(i + PFD < ITERS)
            def _prefetch():
                bn = lax.rem(i + PFD, NB)

                @pl.when(i + PFD >= NB)
                def _slot_free():
                    pltpu.make_async_copy(
                        rows_v.at[bn], accum.at[didx_v.at[i]], ssem.at[bn]
                    ).wait()

                pltpu.async_copy(
                    y_hbm.at[sidx_v.at[i + PFD]],
                    rows_v.at[bn], gsem.at[bn])

            b = lax.rem(i, NB)
            pltpu.make_async_copy(
                y_hbm.at[sidx_v.at[i]], rows_v.at[b], gsem.at[b]).wait()
            pltpu.async_copy(rows_v.at[b], accum.at[didx_v.at[i]],
                             ssem.at[b], add=True)
            return carry

        lax.fori_loop(0, ITERS, body, 0)
        for j in range(NB):
            pltpu.make_async_copy(
                rows_v.at[j], accum.at[didx_v.at[0]], ssem.at[j]).wait()
        plsc.subcore_barrier()

        @pl.when(sid == 0)
        def _writeback():
            pltpu.sync_copy(accum, out_hbm.at[cid])

    return k(src3, dst3, y)


_BLK = 1000


def _tc_matmul(x, W1):
    def body(x_ref, w_ref, o_ref):
        o_ref[...] = jnp.dot(x_ref[...], w_ref[...],
                             preferred_element_type=jnp.float32)

    return pl.pallas_call(
        body,
        grid=(N // _BLK,),
        in_specs=[
            pl.BlockSpec((_BLK, IN), lambda i: (i, 0)),
            pl.BlockSpec((IN, H1), lambda i: (0, 0)),
        ],
        out_specs=pl.BlockSpec((_BLK, H1), lambda i: (i, 0)),
        out_shape=jax.ShapeDtypeStruct((N, H1), jnp.float32),
    )(x, W1)


def _tc_scale(dp, xw1):
    def body(dp_ref, xw_ref, y_ref, dinv_ref):
        deg = dp_ref[0, :, 0:1] + dp_ref[1, :, 0:1] + 1.0
        dinv = lax.rsqrt(deg)
        y_ref[...] = dinv * xw_ref[...]
        dinv_ref[...] = jnp.broadcast_to(dinv, (_BLK, 16))

    return pl.pallas_call(
        body,
        grid=(N // _BLK,),
        in_specs=[
            pl.BlockSpec((NC, _BLK, 16), lambda i: (0, i, 0)),
            pl.BlockSpec((_BLK, H1), lambda i: (i, 0)),
        ],
        out_specs=[
            pl.BlockSpec((_BLK, H1), lambda i: (i, 0)),
            pl.BlockSpec((_BLK, 16), lambda i: (i, 0)),
        ],
        out_shape=[
            jax.ShapeDtypeStruct((N, H1), jnp.float32),
            jax.ShapeDtypeStruct((N, 16), jnp.float32),
        ],
    )(dp, xw1)


def _tc_mid(p, y1, dinv16, W2, b1):
    def body(p_ref, y_ref, dinv_ref, w_ref, b_ref, out_ref):
        dinv = dinv_ref[:, 0:1]
        h1 = dinv * (p_ref[0] + p_ref[1] - y_ref[...]) + b_ref[0, :]
        out_ref[...] = dinv * jnp.dot(h1, w_ref[...], preferred_element_type=jnp.float32)

    return pl.pallas_call(
        body,
        grid=(N // _BLK,),
        in_specs=[
            pl.BlockSpec((NC, _BLK, H1), lambda i: (0, i, 0)),
            pl.BlockSpec((_BLK, H1), lambda i: (i, 0)),
            pl.BlockSpec((_BLK, 16), lambda i: (i, 0)),
            pl.BlockSpec((H1, H2), lambda i: (0, 0)),
            pl.BlockSpec((1, H1), lambda i: (0, 0)),
        ],
        out_specs=pl.BlockSpec((_BLK, H2), lambda i: (i, 0)),
        out_shape=jax.ShapeDtypeStruct((N, H2), jnp.float32),
    )(p, y1, dinv16, W2, b1.reshape(1, H1))


def _tc_heads(r, y2, dinv16, b2, Wmu, bmu, Wlv, blv):
    def body(r_ref, y_ref, dinv_ref, b2_ref, wmu_ref, bmu_ref, wlv_ref, blv_ref,
             mu_ref, lv_ref):
        dinv = dinv_ref[:, 0:1]
        h2 = dinv * (r_ref[0] + r_ref[1] - y_ref[...]) + b2_ref[0, :]
        mu_ref[...] = jnp.dot(h2, wmu_ref[...], preferred_element_type=jnp.float32) + bmu_ref[0, :]
        lv_ref[...] = jnp.dot(h2, wlv_ref[...], preferred_element_type=jnp.float32) + blv_ref[0, :]

    return pl.pallas_call(
        body,
        grid=(N // _BLK,),
        in_specs=[
            pl.BlockSpec((NC, _BLK, H2), lambda i: (0, i, 0)),
            pl.BlockSpec((_BLK, H2), lambda i: (i, 0)),
            pl.BlockSpec((_BLK, 16), lambda i: (i, 0)),
            pl.BlockSpec((1, H2), lambda i: (0, 0)),
            pl.BlockSpec((H2, OUT), lambda i: (0, 0)),
            pl.BlockSpec((1, OUT), lambda i: (0, 0)),
            pl.BlockSpec((H2, OUT), lambda i: (0, 0)),
            pl.BlockSpec((1, OUT), lambda i: (0, 0)),
        ],
        out_specs=[
            pl.BlockSpec((_BLK, OUT), lambda i: (i, 0)),
            pl.BlockSpec((_BLK, OUT), lambda i: (i, 0)),
        ],
        out_shape=[
            jax.ShapeDtypeStruct((N, OUT), jnp.float32),
            jax.ShapeDtypeStruct((N, OUT), jnp.float32),
        ],
    )(r, y2, dinv16, b2.reshape(1, H2), Wmu, bmu.reshape(1, OUT),
      Wlv, blv.reshape(1, OUT))


def kernel(x, edge_index, W1, b1, W2, b2, Wmu, bmu, Wlv, blv):
    src3 = lax.optimization_barrier(edge_index[0]).reshape(NW, ITERS, CHUNK)
    dst3 = edge_index[1].reshape(NW, ITERS, CHUNK)
    xw1 = _tc_matmul(x, W1)
    dp = _sc_degree(dst3, jnp.zeros((N, 16), jnp.float32))
    y1, dinv16 = _tc_scale(dp, xw1)
    p = _sc_propagate(src3, dst3, y1)
    y2 = _tc_mid(p, y1, dinv16, W2, b1)
    r = _sc_propagate(src3, dst3, y2)
    mu, log_var = _tc_heads(r, y2, dinv16, b2, Wmu, bmu, Wlv, blv)
    return (mu, log_var)
